# baseline (device time: 37201 ns/iter reference)
import jax
import jax.numpy as jnp
from jax import lax
from jax.experimental import pallas as pl
from jax.experimental.pallas import tpu as pltpu

IV = {
    "Y1": (0, 208), "Y2": (208, 408), "X1": (408, 616),
    "X2": (616, 816), "R": (816, 1024),
}
HV = {
    "Y1": ((0, 104), (104, 208)),
    "Y2": ((208, 304), (304, 408)),
    "X1": ((408, 512), (512, 616)),
    "X2": ((616, 712), (712, 816)),
    "R": ((816, 920), (920, 1024)),
}

TBL = {
    (0, 0): dict(own=["Y1", "Y2"], xs="Y2", xr="X2",
                 zms=[], zps=["Y2"], zmr=[], zpr=["X1", "R"]),
    (1, 0): dict(own=["X1", "X2"], xs="X2", xr="Y2",
                 zms=[], zps=["X2"], zmr=[], zpr=["Y1", "R"]),
    (0, 1): dict(own=["X1", "R"], xs="X1", xr="Y1",
                 zms=["X1", "R"], zps=["X1"], zmr=["Y2"], zpr=["X2"]),
    (1, 1): dict(own=["Y1", "R"], xs="Y1", xr="X1",
                 zms=["Y1", "R"], zps=["Y1"], zmr=["X2"], zpr=["Y2"]),
    (0, 2): dict(own=["X2", "R"], xs="X2", xr="Y2",
                 zms=["X2"], zps=["X2", "R"], zmr=["X1"], zpr=["Y1"]),
    (1, 2): dict(own=["Y2", "R"], xs="Y2", xr="X2",
                 zms=["Y2"], zps=["Y2", "R"], zmr=["Y1"], zpr=["X1"]),
    (0, 3): dict(own=["Y1", "Y2"], xs="Y1", xr="X1",
                 zms=["Y1"], zps=[], zmr=["X2", "R"], zpr=[]),
    (1, 3): dict(own=["X1", "X2"], xs="X1", xr="Y1",
                 zms=["X1"], zps=[], zmr=["Y2", "R"], zpr=[]),
}

YB_OFF = 208


def kernel(partial, resid, gamma):
    _, m, d = partial.shape
    p2 = partial.reshape(m, d)
    g2 = gamma.reshape(1, d)

    def body(p_ref, r_ref, g_ref, o_ref, ybuf,
             ys, yr, xs, xr, zms, zmr, zps, zpr):
        my_x = lax.axis_index("x")
        my_y = lax.axis_index("y")
        my_z = lax.axis_index("z")
        py = (my_x, 1 - my_y, my_z)
        px = (1 - my_x, my_y, my_z)

        MESH = pl.DeviceIdType.MESH

        def branch(X, Z, t):
            zm = (my_x, my_y, Z - 1)
            zp = (my_x, my_y, Z + 1)

            nbrs = [py, px]
            if Z > 0:
                nbrs.append(zm)
            if Z < 3:
                nbrs.append(zp)
            barrier_sem = pltpu.get_barrier_semaphore()
            for nbr in nbrs:
                pl.semaphore_signal(barrier_sem, inc=1, device_id=nbr,
                                    device_id_type=MESH)
            pl.semaphore_wait(barrier_sem, len(nbrs))

            pend = []
            ywait = []
            for j, iv in enumerate(t["own"]):
                lo0 = IV[iv][0]
                for h, (lo, hi) in enumerate(HV[iv]):
                    boff = j * YB_OFF + (lo - lo0)
                    r = pltpu.make_async_remote_copy(
                        src_ref=p_ref.at[pl.ds(lo, hi - lo), :],
                        dst_ref=ybuf.at[pl.ds(boff, hi - lo), :],
                        send_sem=ys.at[2 * j + h], recv_sem=yr.at[2 * j + h],
                        device_id=py, device_id_type=MESH,
                    )
                    r.start()
                    ywait.append(r)
                    pend.append(r)

            def sends_for(iv):
                out = []
                if t["xs"] == iv:
                    out.append((px, xs, xr, 0))
                for k, f in enumerate(t["zms"]):
                    if f == iv:
                        out.append((zm, zms, zpr, 2 * k))
                for k, f in enumerate(t["zps"]):
                    if f == iv:
                        out.append((zp, zps, zmr, 2 * k))
                return out

            for j, iv in enumerate(t["own"]):
                lo0 = IV[iv][0]
                flows = sends_for(iv)
                for h, (lo, hi) in enumerate(HV[iv]):
                    sz = hi - lo
                    boff = j * YB_OFF + (lo - lo0)
                    ywait[2 * j + h].wait_recv()
                    rows = pl.ds(lo, sz)
                    y = p_ref[rows, :] + ybuf[pl.ds(boff, sz), :] + r_ref[rows, :]
                    rms = jnp.sqrt(
                        jnp.mean(y * y, axis=-1, keepdims=True) + 1e-6)
                    o_ref[rows, :] = y / rms * g_ref[...]
                    for dev, ssem, rsem, k0 in flows:
                        r = pltpu.make_async_remote_copy(
                            src_ref=o_ref.at[rows, :],
                            dst_ref=o_ref.at[rows, :],
                            send_sem=ssem.at[k0 + h], recv_sem=rsem.at[k0 + h],
                            device_id=dev, device_id_type=MESH,
                        )
                        r.start()
                        pend.append(r)

            recvs = [(t["xr"], xr, 0, px)]
            for k, f in enumerate(t["zmr"]):
                recvs.append((f, zmr, 2 * k, zm))
            for k, f in enumerate(t["zpr"]):
                recvs.append((f, zpr, 2 * k, zp))
            for iv, rsem, k0, dev in recvs:
                for h, (lo, hi) in enumerate(HV[iv]):
                    rows = pl.ds(lo, hi - lo)
                    pltpu.make_async_remote_copy(
                        src_ref=o_ref.at[rows, :], dst_ref=o_ref.at[rows, :],
                        send_sem=rsem.at[k0 + h], recv_sem=rsem.at[k0 + h],
                        device_id=dev, device_id_type=MESH,
                    ).wait_recv()

            for r in pend:
                r.wait_send()

        for (X, Z), t in TBL.items():
            @pl.when(jnp.logical_and(my_x == X, my_z == Z))
            def _(X=X, Z=Z, t=t):
                branch(X, Z, t)

    return pl.pallas_call(
        body,
        out_shape=jax.ShapeDtypeStruct((m, d), jnp.float32),
        in_specs=[
            pl.BlockSpec(memory_space=pltpu.VMEM),
            pl.BlockSpec(memory_space=pltpu.VMEM),
            pl.BlockSpec(memory_space=pltpu.VMEM),
        ],
        out_specs=pl.BlockSpec(memory_space=pltpu.VMEM),
        scratch_shapes=[
            pltpu.VMEM((2 * YB_OFF, d), jnp.float32),
            pltpu.SemaphoreType.DMA((4,)),
            pltpu.SemaphoreType.DMA((4,)),
            pltpu.SemaphoreType.DMA((2,)),
            pltpu.SemaphoreType.DMA((2,)),
            pltpu.SemaphoreType.DMA((4,)),
            pltpu.SemaphoreType.DMA((4,)),
            pltpu.SemaphoreType.DMA((4,)),
            pltpu.SemaphoreType.DMA((4,)),
        ],
        compiler_params=pltpu.CompilerParams(collective_id=0),
    )(p2, resid, g2)


# device time: 35322 ns/iter; 1.0532x vs baseline; 1.0532x over previous
import jax
import jax.numpy as jnp
from jax import lax
from jax.experimental import pallas as pl
from jax.experimental.pallas import tpu as pltpu

IV = {
    "Y1": (0, 208), "Y2": (208, 408), "X1": (408, 616),
    "X2": (616, 816), "R": (816, 1024),
}
NH = 4


def _cuts(lo, hi):
    sz = hi - lo
    pts = [lo + (sz * k // NH) // 8 * 8 for k in range(NH)] + [hi]
    return tuple((pts[k], pts[k + 1]) for k in range(NH))


HV = {name: _cuts(lo, hi) for name, (lo, hi) in IV.items()}

TBL = {
    (0, 0): dict(own=["Y1", "Y2"], xs="Y2", xr="X2",
                 zms=[], zps=["Y2"], zmr=[], zpr=["X1", "R"]),
    (1, 0): dict(own=["X1", "X2"], xs="X2", xr="Y2",
                 zms=[], zps=["X2"], zmr=[], zpr=["Y1", "R"]),
    (0, 1): dict(own=["X1", "R"], xs="X1", xr="Y1",
                 zms=["X1", "R"], zps=["X1"], zmr=["Y2"], zpr=["X2"]),
    (1, 1): dict(own=["Y1", "R"], xs="Y1", xr="X1",
                 zms=["Y1", "R"], zps=["Y1"], zmr=["X2"], zpr=["Y2"]),
    (0, 2): dict(own=["X2", "R"], xs="X2", xr="Y2",
                 zms=["X2"], zps=["X2", "R"], zmr=["X1"], zpr=["Y1"]),
    (1, 2): dict(own=["Y2", "R"], xs="Y2", xr="X2",
                 zms=["Y2"], zps=["Y2", "R"], zmr=["Y1"], zpr=["X1"]),
    (0, 3): dict(own=["Y1", "Y2"], xs="Y1", xr="X1",
                 zms=["Y1"], zps=[], zmr=["X2", "R"], zpr=[]),
    (1, 3): dict(own=["X1", "X2"], xs="X1", xr="Y1",
                 zms=["X1"], zps=[], zmr=["Y2", "R"], zpr=[]),
}

YB_OFF = 208


def kernel(partial, resid, gamma):
    _, m, d = partial.shape
    p2 = partial.reshape(m, d)
    g2 = gamma.reshape(1, d)

    def body(p_ref, r_ref, g_ref, o_ref, ybuf,
             ys, yr, xs, xr, zms, zmr, zps, zpr):
        my_x = lax.axis_index("x")
        my_y = lax.axis_index("y")
        my_z = lax.axis_index("z")
        py = (my_x, 1 - my_y, my_z)
        px = (1 - my_x, my_y, my_z)

        MESH = pl.DeviceIdType.MESH

        def branch(X, Z, t):
            zm = (my_x, my_y, Z - 1)
            zp = (my_x, my_y, Z + 1)

            nbrs = [py, px]
            if Z > 0:
                nbrs.append(zm)
            if Z < 3:
                nbrs.append(zp)
            barrier_sem = pltpu.get_barrier_semaphore()
            for nbr in nbrs:
                pl.semaphore_signal(barrier_sem, inc=1, device_id=nbr,
                                    device_id_type=MESH)
            pl.semaphore_wait(barrier_sem, len(nbrs))

            pend = []
            ywait = []
            for j, iv in enumerate(t["own"]):
                lo0 = IV[iv][0]
                for h, (lo, hi) in enumerate(HV[iv]):
                    boff = j * YB_OFF + (lo - lo0)
                    r = pltpu.make_async_remote_copy(
                        src_ref=p_ref.at[pl.ds(lo, hi - lo), :],
                        dst_ref=ybuf.at[pl.ds(boff, hi - lo), :],
                        send_sem=ys.at[NH * j + h], recv_sem=yr.at[NH * j + h],
                        device_id=py, device_id_type=MESH,
                    )
                    r.start()
                    ywait.append(r)
                    pend.append(r)

            def sends_for(iv):
                out = []
                if t["xs"] == iv:
                    out.append((px, xs, xr, 0))
                for k, f in enumerate(t["zms"]):
                    if f == iv:
                        out.append((zm, zms, zpr, NH * k))
                for k, f in enumerate(t["zps"]):
                    if f == iv:
                        out.append((zp, zps, zmr, NH * k))
                return out

            for j, iv in enumerate(t["own"]):
                lo0 = IV[iv][0]
                flows = sends_for(iv)
                for h, (lo, hi) in enumerate(HV[iv]):
                    sz = hi - lo
                    boff = j * YB_OFF + (lo - lo0)
                    ywait[NH * j + h].wait_recv()
                    rows = pl.ds(lo, sz)
                    y = p_ref[rows, :] + ybuf[pl.ds(boff, sz), :] + r_ref[rows, :]
                    rms = jnp.sqrt(
                        jnp.mean(y * y, axis=-1, keepdims=True) + 1e-6)
                    o_ref[rows, :] = y / rms * g_ref[...]
                    for dev, ssem, rsem, k0 in flows:
                        r = pltpu.make_async_remote_copy(
                            src_ref=o_ref.at[rows, :],
                            dst_ref=o_ref.at[rows, :],
                            send_sem=ssem.at[k0 + h], recv_sem=rsem.at[k0 + h],
                            device_id=dev, device_id_type=MESH,
                        )
                        r.start()
                        pend.append(r)

            recvs = [(t["xr"], xr, 0, px)]
            for k, f in enumerate(t["zmr"]):
                recvs.append((f, zmr, NH * k, zm))
            for k, f in enumerate(t["zpr"]):
                recvs.append((f, zpr, NH * k, zp))
            for iv, rsem, k0, dev in recvs:
                for h, (lo, hi) in enumerate(HV[iv]):
                    rows = pl.ds(lo, hi - lo)
                    pltpu.make_async_remote_copy(
                        src_ref=o_ref.at[rows, :], dst_ref=o_ref.at[rows, :],
                        send_sem=rsem.at[k0 + h], recv_sem=rsem.at[k0 + h],
                        device_id=dev, device_id_type=MESH,
                    ).wait_recv()

            for r in pend:
                r.wait_send()

        for (X, Z), t in TBL.items():
            @pl.when(jnp.logical_and(my_x == X, my_z == Z))
            def _(X=X, Z=Z, t=t):
                branch(X, Z, t)

    return pl.pallas_call(
        body,
        out_shape=jax.ShapeDtypeStruct((m, d), jnp.float32),
        in_specs=[
            pl.BlockSpec(memory_space=pltpu.VMEM),
            pl.BlockSpec(memory_space=pltpu.VMEM),
            pl.BlockSpec(memory_space=pltpu.VMEM),
        ],
        out_specs=pl.BlockSpec(memory_space=pltpu.VMEM),
        scratch_shapes=[
            pltpu.VMEM((2 * YB_OFF, d), jnp.float32),
            pltpu.SemaphoreType.DMA((2 * NH,)),
            pltpu.SemaphoreType.DMA((2 * NH,)),
            pltpu.SemaphoreType.DMA((NH,)),
            pltpu.SemaphoreType.DMA((NH,)),
            pltpu.SemaphoreType.DMA((2 * NH,)),
            pltpu.SemaphoreType.DMA((2 * NH,)),
            pltpu.SemaphoreType.DMA((2 * NH,)),
            pltpu.SemaphoreType.DMA((2 * NH,)),
        ],
        compiler_params=pltpu.CompilerParams(collective_id=0),
    )(p2, resid, g2)


# device time: 34791 ns/iter; 1.0693x vs baseline; 1.0153x over previous
import jax
import jax.numpy as jnp
from jax import lax
from jax.experimental import pallas as pl
from jax.experimental.pallas import tpu as pltpu

IV = {
    "Y1": (0, 208), "Y2": (208, 408), "X1": (408, 616),
    "X2": (616, 816), "R": (816, 1024),
}
NH = 8


def _cuts(lo, hi):
    sz = hi - lo
    pts = [lo + (sz * k // NH) // 8 * 8 for k in range(NH)] + [hi]
    return tuple((pts[k], pts[k + 1]) for k in range(NH))


HV = {name: _cuts(lo, hi) for name, (lo, hi) in IV.items()}

TBL = {
    (0, 0): dict(own=["Y1", "Y2"], xs="Y2", xr="X2",
                 zms=[], zps=["Y2"], zmr=[], zpr=["X1", "R"]),
    (1, 0): dict(own=["X1", "X2"], xs="X2", xr="Y2",
                 zms=[], zps=["X2"], zmr=[], zpr=["Y1", "R"]),
    (0, 1): dict(own=["X1", "R"], xs="X1", xr="Y1",
                 zms=["X1", "R"], zps=["X1"], zmr=["Y2"], zpr=["X2"]),
    (1, 1): dict(own=["Y1", "R"], xs="Y1", xr="X1",
                 zms=["Y1", "R"], zps=["Y1"], zmr=["X2"], zpr=["Y2"]),
    (0, 2): dict(own=["X2", "R"], xs="X2", xr="Y2",
                 zms=["X2"], zps=["X2", "R"], zmr=["X1"], zpr=["Y1"]),
    (1, 2): dict(own=["Y2", "R"], xs="Y2", xr="X2",
                 zms=["Y2"], zps=["Y2", "R"], zmr=["Y1"], zpr=["X1"]),
    (0, 3): dict(own=["Y1", "Y2"], xs="Y1", xr="X1",
                 zms=["Y1"], zps=[], zmr=["X2", "R"], zpr=[]),
    (1, 3): dict(own=["X1", "X2"], xs="X1", xr="Y1",
                 zms=["X1"], zps=[], zmr=["Y2", "R"], zpr=[]),
}

YB_OFF = 208


def kernel(partial, resid, gamma):
    _, m, d = partial.shape
    p2 = partial.reshape(m, d)
    g2 = gamma.reshape(1, d)

    def body(p_ref, r_ref, g_ref, o_ref, ybuf,
             ys, yr, xs, xr, zms, zmr, zps, zpr):
        my_x = lax.axis_index("x")
        my_y = lax.axis_index("y")
        my_z = lax.axis_index("z")
        py = (my_x, 1 - my_y, my_z)
        px = (1 - my_x, my_y, my_z)

        MESH = pl.DeviceIdType.MESH

        def branch(X, Z, t):
            zm = (my_x, my_y, Z - 1)
            zp = (my_x, my_y, Z + 1)

            nbrs = [py, px]
            if Z > 0:
                nbrs.append(zm)
            if Z < 3:
                nbrs.append(zp)
            barrier_sem = pltpu.get_barrier_semaphore()
            for nbr in nbrs:
                pl.semaphore_signal(barrier_sem, inc=1, device_id=nbr,
                                    device_id_type=MESH)
            pl.semaphore_wait(barrier_sem, len(nbrs))

            pend = []
            ywait = []
            for j, iv in enumerate(t["own"]):
                lo0 = IV[iv][0]
                for h, (lo, hi) in enumerate(HV[iv]):
                    boff = j * YB_OFF + (lo - lo0)
                    r = pltpu.make_async_remote_copy(
                        src_ref=p_ref.at[pl.ds(lo, hi - lo), :],
                        dst_ref=ybuf.at[pl.ds(boff, hi - lo), :],
                        send_sem=ys.at[NH * j + h], recv_sem=yr.at[NH * j + h],
                        device_id=py, device_id_type=MESH,
                    )
                    r.start()
                    ywait.append(r)
                    pend.append(r)

            def sends_for(iv):
                out = []
                if t["xs"] == iv:
                    out.append((px, xs, xr, 0))
                for k, f in enumerate(t["zms"]):
                    if f == iv:
                        out.append((zm, zms, zpr, NH * k))
                for k, f in enumerate(t["zps"]):
                    if f == iv:
                        out.append((zp, zps, zmr, NH * k))
                return out

            for j, iv in enumerate(t["own"]):
                lo0 = IV[iv][0]
                flows = sends_for(iv)
                for h, (lo, hi) in enumerate(HV[iv]):
                    sz = hi - lo
                    boff = j * YB_OFF + (lo - lo0)
                    ywait[NH * j + h].wait_recv()
                    rows = pl.ds(lo, sz)
                    y = p_ref[rows, :] + ybuf[pl.ds(boff, sz), :] + r_ref[rows, :]
                    rms = jnp.sqrt(
                        jnp.mean(y * y, axis=-1, keepdims=True) + 1e-6)
                    o_ref[rows, :] = y / rms * g_ref[...]
                    for dev, ssem, rsem, k0 in flows:
                        r = pltpu.make_async_remote_copy(
                            src_ref=o_ref.at[rows, :],
                            dst_ref=o_ref.at[rows, :],
                            send_sem=ssem.at[k0 + h], recv_sem=rsem.at[k0 + h],
                            device_id=dev, device_id_type=MESH,
                        )
                        r.start()
                        pend.append(r)

            recvs = [(t["xr"], xr, 0, px)]
            for k, f in enumerate(t["zmr"]):
                recvs.append((f, zmr, NH * k, zm))
            for k, f in enumerate(t["zpr"]):
                recvs.append((f, zpr, NH * k, zp))
            for iv, rsem, k0, dev in recvs:
                for h, (lo, hi) in enumerate(HV[iv]):
                    rows = pl.ds(lo, hi - lo)
                    pltpu.make_async_remote_copy(
                        src_ref=o_ref.at[rows, :], dst_ref=o_ref.at[rows, :],
                        send_sem=rsem.at[k0 + h], recv_sem=rsem.at[k0 + h],
                        device_id=dev, device_id_type=MESH,
                    ).wait_recv()

            for r in pend:
                r.wait_send()

        for (X, Z), t in TBL.items():
            @pl.when(jnp.logical_and(my_x == X, my_z == Z))
            def _(X=X, Z=Z, t=t):
                branch(X, Z, t)

    return pl.pallas_call(
        body,
        out_shape=jax.ShapeDtypeStruct((m, d), jnp.float32),
        in_specs=[
            pl.BlockSpec(memory_space=pltpu.VMEM),
            pl.BlockSpec(memory_space=pltpu.VMEM),
            pl.BlockSpec(memory_space=pltpu.VMEM),
        ],
        out_specs=pl.BlockSpec(memory_space=pltpu.VMEM),
        scratch_shapes=[
            pltpu.VMEM((2 * YB_OFF, d), jnp.float32),
            pltpu.SemaphoreType.DMA((2 * NH,)),
            pltpu.SemaphoreType.DMA((2 * NH,)),
            pltpu.SemaphoreType.DMA((NH,)),
            pltpu.SemaphoreType.DMA((NH,)),
            pltpu.SemaphoreType.DMA((2 * NH,)),
            pltpu.SemaphoreType.DMA((2 * NH,)),
            pltpu.SemaphoreType.DMA((2 * NH,)),
            pltpu.SemaphoreType.DMA((2 * NH,)),
        ],
        compiler_params=pltpu.CompilerParams(collective_id=0),
    )(p2, resid, g2)
